# baseline (speedup 1.0000x reference)
import functools

import jax
import jax.numpy as jnp
from jax.experimental import pallas as pl
from jax.experimental.pallas import tpu as pltpu

_NBUF = 4
_BM = 200


def _gcn_body(a_hbm, x_ref, w_ref, o_ref, a_buf, sems, *, nsteps):
    def copy_in(i, slot):
        return pltpu.make_async_copy(
            a_hbm.at[pl.ds(i * _BM, _BM), :], a_buf.at[slot], sems.at[slot]
        )

    for j in range(_NBUF):
        copy_in(j, j).start()

    x_blk = x_ref[...].astype(jnp.bfloat16)
    w_row = w_ref[...]

    def step(i, carry):
        slot = jax.lax.rem(i, _NBUF)
        copy_in(i, slot).wait()
        acc = jnp.dot(
            a_buf[slot].astype(jnp.bfloat16),
            x_blk,
            preferred_element_type=jnp.float32,
        )
        o_ref[pl.ds(i * _BM, _BM), :] = acc * w_row

        @pl.when(i + _NBUF < nsteps)
        def _prefetch():
            copy_in(i + _NBUF, slot).start()

        return carry

    jax.lax.fori_loop(0, nsteps, step, 0)


def kernel(input, A, W):
    n, d = input.shape
    w2d = W.reshape(1, d)
    return pl.pallas_call(
        functools.partial(_gcn_body, nsteps=n // _BM),
        in_specs=[
            pl.BlockSpec(memory_space=pltpu.MemorySpace.HBM),
            pl.BlockSpec(memory_space=pltpu.MemorySpace.VMEM),
            pl.BlockSpec(memory_space=pltpu.MemorySpace.VMEM),
        ],
        out_specs=pl.BlockSpec(memory_space=pltpu.MemorySpace.VMEM),
        out_shape=jax.ShapeDtypeStruct((n, d), jnp.float32),
        scratch_shapes=[
            pltpu.VMEM((_NBUF, _BM, n), jnp.float32),
            pltpu.SemaphoreType.DMA((_NBUF,)),
        ],
    )(A, input, w2d)
